# baseline (speedup 1.0000x reference)
import jax
import jax.numpy as jnp
from jax.experimental import pallas as pl

N_NODES = 10000


def _lsm_body(h_ref, o_ref):
    v = h_ref[...]
    m = jnp.max(v, axis=-1, keepdims=True)
    z = v - m
    lse = jnp.log(jnp.sum(jnp.exp(z), axis=-1, keepdims=True))
    o_ref[...] = z - lse


def _sage(x, src, dst, W_l, b, W_r, n_nodes):
    msg = jnp.take(x, src, axis=0)
    agg_sum = jnp.zeros((n_nodes, x.shape[1]), dtype=x.dtype).at[dst].add(msg)
    deg = jnp.zeros((n_nodes,), dtype=x.dtype).at[dst].add(1.0)
    deg = jnp.maximum(deg, 1.0)
    agg = agg_sum / deg[:, None]
    return agg @ W_l.T + b + x @ W_r.T


def kernel(x, edge_index, W1_l, b1, W1_r, W2_l, b2, W2_r):
    src = edge_index[0]
    dst = edge_index[1]
    h = jax.nn.relu(_sage(x, src, dst, W1_l, b1, W1_r, N_NODES))
    h2 = _sage(h, src, dst, W2_l, b2, W2_r, N_NODES)
    BN = 400
    out = pl.pallas_call(
        _lsm_body,
        grid=(N_NODES // BN,),
        in_specs=[pl.BlockSpec((BN, 64), lambda i: (i, 0))],
        out_specs=pl.BlockSpec((BN, 64), lambda i: (i, 0)),
        out_shape=jax.ShapeDtypeStruct((N_NODES, 64), jnp.float32),
    )(h2)
    return out


# speedup vs baseline: 4.0954x; 4.0954x over previous
import functools

import jax
import jax.numpy as jnp
from jax import lax
from jax.experimental import pallas as pl
from jax.experimental.pallas import tpu as pltpu
from jax.experimental.pallas import tpu_sc as plsc

N = 10000
E = 320000
NC = 2
NS = 16
NW = NC * NS
CHUNK = 128
NCHUNK = 80
E_PAD = NW * NCHUNK * CHUNK
N_PAD = 10240
ROWS_PER_TILE = N_PAD // NS
D_AUG = 144
BN = 512


def _make_sc_aggregate(D):
    mesh = plsc.VectorSubcoreMesh(core_axis_name="c", subcore_axis_name="s")
    scratch = [
        pltpu.VMEM((NCHUNK, CHUNK), jnp.int32),
        pltpu.VMEM((NCHUNK, CHUNK), jnp.int32),
        pltpu.VMEM((CHUNK, D), jnp.float32),
        pltpu.VMEM_SHARED((N_PAD, D), jnp.float32),
        pltpu.SemaphoreType.DMA,
    ]

    @functools.partial(
        pl.kernel, mesh=mesh,
        out_type=jax.ShapeDtypeStruct((NC, N_PAD, D), jnp.float32),
        scratch_types=scratch,
        compiler_params=pltpu.CompilerParams(use_tc_tiling_on_sc=False))
    def agg(vals, src3, dst3, out, src_v, dst_v, rows, acc, sem):
        c = lax.axis_index("c")
        s = lax.axis_index("s")
        wid = c * NS + s
        r0 = s * ROWS_PER_TILE

        zero16 = jnp.zeros((16,), jnp.float32)

        def zrow(r, _):
            def zcol(k, _):
                rows[r, pl.ds(k * 16, 16)] = zero16
                return 0
            return lax.fori_loop(0, D // 16, zcol, 0)
        lax.fori_loop(0, CHUNK, zrow, 0)
        for b in range(ROWS_PER_TILE // CHUNK):
            pltpu.sync_copy(rows, acc.at[pl.ds(r0 + b * CHUNK, CHUNK)])

        pltpu.sync_copy(src3.at[wid], src_v)
        pltpu.sync_copy(dst3.at[wid], dst_v)

        plsc.subcore_barrier()

        def body(j, _):
            pltpu.async_copy(vals.at[src_v.at[j]], rows, sem).wait()
            pltpu.sync_copy(rows, acc.at[dst_v.at[j]], add=True)
            return 0
        lax.fori_loop(0, NCHUNK, body, 0)

        plsc.subcore_barrier()

        pltpu.sync_copy(acc.at[pl.ds(r0, ROWS_PER_TILE)],
                        out.at[c, pl.ds(r0, ROWS_PER_TILE)])

    return agg


_sc_agg_l1 = _make_sc_aggregate(D_AUG)
_sc_agg_l2 = _make_sc_aggregate(64)


def _tc_layer1_body(acc_ref, x_ref, w1lt_ref, b1_ref, w1rt_ref,
                    w2lt_ref, w2rt_ref, y2_ref, hr_ref, deg_ref):
    a = acc_ref[0] + acc_ref[1]
    deg = jnp.maximum(a[:, 128], 1.0)
    agg = a[:, :128] / deg[:, None]
    h = agg @ w1lt_ref[...] + b1_ref[...] + x_ref[...] @ w1rt_ref[...]
    h = jnp.maximum(h, 0.0)
    y2_ref[...] = h @ w2lt_ref[...]
    hr_ref[...] = h @ w2rt_ref[...]
    deg_ref[...] = jnp.broadcast_to(deg[:, None], deg_ref.shape)


def _tc_layer2_body(acc_ref, deg_ref, hr_ref, b2_ref, o_ref):
    a = acc_ref[0] + acc_ref[1]
    v = a / deg_ref[:, :1] + b2_ref[...] + hr_ref[...]
    z = v - jnp.max(v, axis=-1, keepdims=True)
    o_ref[...] = z - jnp.log(jnp.sum(jnp.exp(z), axis=-1, keepdims=True))


def kernel(x, edge_index, W1_l, b1, W1_r, W2_l, b2, W2_r):
    src = edge_index[0].astype(jnp.int32)
    dst = edge_index[1].astype(jnp.int32)
    pad = E_PAD - E
    src3 = jnp.concatenate([src, jnp.zeros((pad,), jnp.int32)]).reshape(
        NW, NCHUNK, CHUNK)
    dst3 = jnp.concatenate([dst, jnp.full((pad,), N, jnp.int32)]).reshape(
        NW, NCHUNK, CHUNK)

    ones_col = jnp.ones((N, 1), jnp.float32)
    x_aug = jnp.concatenate(
        [x, ones_col, jnp.zeros((N, D_AUG - 129), jnp.float32)], axis=1)

    acc1 = _sc_agg_l1(x_aug, src3, dst3)

    x_pad = jnp.pad(x, ((0, N_PAD - N), (0, 0)))
    grid = (N_PAD // BN,)
    wfull = lambda shp: pl.BlockSpec(shp, lambda i: (0, 0))
    y2, hr, deg8 = pl.pallas_call(
        _tc_layer1_body,
        grid=grid,
        in_specs=[
            pl.BlockSpec((NC, BN, D_AUG), lambda i: (0, i, 0)),
            pl.BlockSpec((BN, 128), lambda i: (i, 0)),
            wfull((128, 128)),
            wfull((1, 128)),
            wfull((128, 128)),
            wfull((128, 64)),
            wfull((128, 64)),
        ],
        out_specs=[pl.BlockSpec((BN, 64), lambda i: (i, 0))] * 2
        + [pl.BlockSpec((BN, 8), lambda i: (i, 0))],
        out_shape=[jax.ShapeDtypeStruct((N_PAD, 64), jnp.float32)] * 2
        + [jax.ShapeDtypeStruct((N_PAD, 8), jnp.float32)],
    )(acc1, x_pad, W1_l.T, b1[None, :], W1_r.T, W2_l.T, W2_r.T)

    acc2 = _sc_agg_l2(y2, src3, dst3)

    out = pl.pallas_call(
        _tc_layer2_body,
        grid=grid,
        in_specs=[
            pl.BlockSpec((NC, BN, 64), lambda i: (0, i, 0)),
            pl.BlockSpec((BN, 8), lambda i: (i, 0)),
            pl.BlockSpec((BN, 64), lambda i: (i, 0)),
            wfull((1, 64)),
        ],
        out_specs=pl.BlockSpec((BN, 64), lambda i: (i, 0)),
        out_shape=jax.ShapeDtypeStruct((N_PAD, 64), jnp.float32),
    )(acc2, deg8, hr, b2[None, :])
    return out[:N]
